# baseline (speedup 1.0000x reference)
import functools

import jax
import jax.numpy as jnp
from jax import lax
from jax.experimental import pallas as pl
from jax.experimental.pallas import tpu as pltpu
from jax.experimental.pallas import tpu_sc as plsc

_NUM_CORES = 2
_NUM_SUBCORES = 16
_NW = _NUM_CORES * _NUM_SUBCORES


def _make_sc_gather(batch, d_model, table_rows):
    feats_per_tile = d_model // _NW
    nvec = batch // 16
    mesh = plsc.VectorSubcoreMesh(
        core_axis_name="c",
        subcore_axis_name="s",
        num_cores=_NUM_CORES,
        num_subcores=_NUM_SUBCORES,
    )

    @functools.partial(
        pl.kernel,
        mesh=mesh,
        out_type=jax.ShapeDtypeStruct((d_model, batch), jnp.float32),
        scratch_types=[
            pltpu.VMEM((table_rows,), jnp.float32),
            pltpu.VMEM((batch,), jnp.int32),
            pltpu.VMEM((batch,), jnp.float32),
            pltpu.SemaphoreType.DMA,
        ],
        compiler_params=pltpu.CompilerParams(
            use_tc_tiling_on_sc=False, needs_layout_passes=False
        ),
    )
    def gather_kernel(table_hbm, idx_hbm, out_hbm, row_v, idx_v, out_v, sem):
        wid = lax.axis_index("s") * _NUM_CORES + lax.axis_index("c")
        pltpu.sync_copy(idx_hbm, idx_v)
        for k in range(feats_per_tile):
            f = wid * feats_per_tile + k
            pltpu.async_copy(table_hbm.at[f], row_v, sem).wait()

            def body(i, carry):
                idx = idx_v[pl.ds(pl.multiple_of(i * 16, 16), 16)]
                out_v[pl.ds(pl.multiple_of(i * 16, 16), 16)] = (
                    plsc.load_gather(row_v, [idx])
                )
                return carry

            lax.fori_loop(0, nvec, body, 0)
            pltpu.sync_copy(out_v, out_hbm.at[f])

    return gather_kernel


def _add_body(g_ref, rel_ref, x_ref, o_ref):
    o_ref[...] = (
        x_ref[...] + g_ref[...][None, :, :] + rel_ref[...][:, :, None]
    )


def kernel(x, timesteps, absolute_pos_embed, relative_pos_embed):
    batch, seq_len, d_model = x.shape

    table_t = absolute_pos_embed.T
    gt = _make_sc_gather(batch, d_model, absolute_pos_embed.shape[0])(
        table_t, timesteps.astype(jnp.int32)
    )

    xt = jnp.transpose(x, (1, 2, 0))
    rel = relative_pos_embed[:seq_len]

    bbl = 128
    out_t = pl.pallas_call(
        _add_body,
        grid=(batch // bbl,),
        in_specs=[
            pl.BlockSpec((d_model, bbl), lambda j: (0, j)),
            pl.BlockSpec((seq_len, d_model), lambda j: (0, 0)),
            pl.BlockSpec((seq_len, d_model, bbl), lambda j: (0, 0, j)),
        ],
        out_specs=pl.BlockSpec((seq_len, d_model, bbl), lambda j: (0, 0, j)),
        out_shape=jax.ShapeDtypeStruct((seq_len, d_model, batch), jnp.float32),
        compiler_params=pltpu.CompilerParams(
            dimension_semantics=("arbitrary",),
            vmem_limit_bytes=100 * 1024 * 1024,
        ),
    )(gt, rel, xt)

    return jnp.transpose(out_t, (2, 0, 1))


# speedup vs baseline: 2.1806x; 2.1806x over previous
import functools

import jax
import jax.numpy as jnp
from jax import lax
from jax.experimental import pallas as pl
from jax.experimental.pallas import tpu as pltpu
from jax.experimental.pallas import tpu_sc as plsc

_NUM_CORES = 2
_NUM_SUBCORES = 16
_NW = _NUM_CORES * _NUM_SUBCORES


def _make_sc_gather(batch, d_model):
    b_per_w = batch // _NW
    mesh = plsc.VectorSubcoreMesh(
        core_axis_name="c",
        subcore_axis_name="s",
        num_cores=_NUM_CORES,
        num_subcores=_NUM_SUBCORES,
    )

    @functools.partial(
        pl.kernel,
        mesh=mesh,
        out_type=jax.ShapeDtypeStruct((batch, d_model), jnp.float32),
        scratch_types=[
            pltpu.VMEM((b_per_w,), jnp.int32),
            pltpu.VMEM((b_per_w, d_model), jnp.float32),
            pltpu.SemaphoreType.DMA,
        ],
        compiler_params=pltpu.CompilerParams(use_tc_tiling_on_sc=False),
    )
    def gather_kernel(table_hbm, idx_hbm, out_hbm, idx_v, rows_v, sem):
        wid = lax.axis_index("s") * _NUM_CORES + lax.axis_index("c")
        base = wid * b_per_w
        pltpu.sync_copy(idx_hbm.at[pl.ds(base, b_per_w)], idx_v)
        pltpu.async_copy(table_hbm.at[idx_v], rows_v, sem).wait()
        pltpu.sync_copy(rows_v, out_hbm.at[pl.ds(base, b_per_w)])

    return gather_kernel


def _add_body(g_ref, rel_ref, x_ref, o_ref):
    o_ref[...] = (
        x_ref[...] + g_ref[...][None, :, :] + rel_ref[...][:, :, None]
    )


def kernel(x, timesteps, absolute_pos_embed, relative_pos_embed):
    batch, seq_len, d_model = x.shape

    table_flat = lax.optimization_barrier(absolute_pos_embed.reshape(-1))
    table_rm = table_flat.reshape(absolute_pos_embed.shape)

    gathered = _make_sc_gather(batch, d_model)(
        table_rm, timesteps.astype(jnp.int32)
    )

    xt = jnp.transpose(x, (1, 2, 0))
    gt = gathered.T
    rel = relative_pos_embed[:seq_len]

    bbl = 128
    out_t = pl.pallas_call(
        _add_body,
        grid=(batch // bbl,),
        in_specs=[
            pl.BlockSpec((d_model, bbl), lambda j: (0, j)),
            pl.BlockSpec((seq_len, d_model), lambda j: (0, 0)),
            pl.BlockSpec((seq_len, d_model, bbl), lambda j: (0, 0, j)),
        ],
        out_specs=pl.BlockSpec((seq_len, d_model, bbl), lambda j: (0, 0, j)),
        out_shape=jax.ShapeDtypeStruct((seq_len, d_model, batch), jnp.float32),
        compiler_params=pltpu.CompilerParams(
            dimension_semantics=("arbitrary",),
            vmem_limit_bytes=100 * 1024 * 1024,
        ),
    )(gt, rel, xt)

    return jnp.transpose(out_t, (2, 0, 1))


# speedup vs baseline: 2.2550x; 1.0341x over previous
import functools

import jax
import jax.numpy as jnp
from jax import lax
from jax.experimental import pallas as pl
from jax.experimental.pallas import tpu as pltpu
from jax.experimental.pallas import tpu_sc as plsc

_NUM_CORES = 2
_NUM_SUBCORES = 16
_NW = _NUM_CORES * _NUM_SUBCORES


def _make_sc_gather(batch, width):
    b_per_w = batch // _NW
    mesh = plsc.VectorSubcoreMesh(
        core_axis_name="c",
        subcore_axis_name="s",
        num_cores=_NUM_CORES,
        num_subcores=_NUM_SUBCORES,
    )

    @functools.partial(
        pl.kernel,
        mesh=mesh,
        out_type=jax.ShapeDtypeStruct((batch, width), jnp.float32),
        scratch_types=[
            pltpu.VMEM((b_per_w,), jnp.int32),
            pltpu.VMEM((b_per_w, width), jnp.float32),
            pltpu.SemaphoreType.DMA,
        ],
        compiler_params=pltpu.CompilerParams(use_tc_tiling_on_sc=True),
    )
    def gather_kernel(table_hbm, idx_hbm, out_hbm, idx_v, rows_v, sem):
        wid = lax.axis_index("s") * _NUM_CORES + lax.axis_index("c")
        base = wid * b_per_w
        pltpu.sync_copy(idx_hbm.at[pl.ds(base, b_per_w)], idx_v)
        pltpu.async_copy(table_hbm.at[idx_v], rows_v, sem).wait()
        pltpu.sync_copy(rows_v, out_hbm.at[pl.ds(base, b_per_w)])

    return gather_kernel


def _add_body(g_ref, rel_ref, x_ref, o_ref):
    o_ref[...] = (
        x_ref[...] + g_ref[...][None, :, :] + rel_ref[...][:, :, None]
    )


def kernel(x, timesteps, absolute_pos_embed, relative_pos_embed):
    batch, seq_len, d_model = x.shape

    table_p = jnp.pad(absolute_pos_embed, ((0, 0), (0, 128 - d_model)))

    gathered = _make_sc_gather(batch, 128)(
        table_p, timesteps.astype(jnp.int32)
    )

    xt = jnp.transpose(x, (1, 2, 0))
    gt = gathered[:, :d_model].T
    rel = relative_pos_embed[:seq_len]

    bbl = 128
    out_t = pl.pallas_call(
        _add_body,
        grid=(batch // bbl,),
        in_specs=[
            pl.BlockSpec((d_model, bbl), lambda j: (0, j)),
            pl.BlockSpec((seq_len, d_model), lambda j: (0, 0)),
            pl.BlockSpec((seq_len, d_model, bbl), lambda j: (0, 0, j)),
        ],
        out_specs=pl.BlockSpec((seq_len, d_model, bbl), lambda j: (0, 0, j)),
        out_shape=jax.ShapeDtypeStruct((seq_len, d_model, batch), jnp.float32),
        compiler_params=pltpu.CompilerParams(
            dimension_semantics=("arbitrary",),
            vmem_limit_bytes=100 * 1024 * 1024,
        ),
    )(gt, rel, xt)

    return jnp.transpose(out_t, (2, 0, 1))


# speedup vs baseline: 2.2808x; 1.0114x over previous
import functools

import jax
import jax.numpy as jnp
from jax import lax
from jax.experimental import pallas as pl
from jax.experimental.pallas import tpu as pltpu
from jax.experimental.pallas import tpu_sc as plsc

_NUM_CORES = 2
_NUM_SUBCORES = 16
_NW = _NUM_CORES * _NUM_SUBCORES


def _make_sc_gather(batch, width):
    b_per_w = batch // _NW
    mesh = plsc.VectorSubcoreMesh(
        core_axis_name="c",
        subcore_axis_name="s",
        num_cores=_NUM_CORES,
        num_subcores=_NUM_SUBCORES,
    )

    @functools.partial(
        pl.kernel,
        mesh=mesh,
        out_type=jax.ShapeDtypeStruct((batch, width), jnp.float32),
        scratch_types=[
            pltpu.VMEM((b_per_w,), jnp.int32),
            pltpu.VMEM((b_per_w, width), jnp.float32),
            pltpu.SemaphoreType.DMA,
        ],
        compiler_params=pltpu.CompilerParams(use_tc_tiling_on_sc=True),
    )
    def gather_kernel(table_hbm, idx_hbm, out_hbm, idx_v, rows_v, sem):
        wid = lax.axis_index("s") * _NUM_CORES + lax.axis_index("c")
        base = wid * b_per_w
        pltpu.sync_copy(idx_hbm.at[pl.ds(base, b_per_w)], idx_v)
        pltpu.async_copy(table_hbm.at[idx_v], rows_v, sem).wait()
        pltpu.sync_copy(rows_v, out_hbm.at[pl.ds(base, b_per_w)])

    return gather_kernel


def _add_body(g_ref, rel_ref, x_ref, o_ref):
    o_ref[...] = (
        x_ref[...] + g_ref[...][None, :, :] + rel_ref[...][:, :, None]
    )


def kernel(x, timesteps, absolute_pos_embed, relative_pos_embed):
    batch, seq_len, d_model = x.shape

    table_p = jnp.pad(absolute_pos_embed, ((0, 0), (0, 128 - d_model)))

    gathered = _make_sc_gather(batch, 128)(
        table_p, timesteps.astype(jnp.int32)
    )

    xt = jnp.transpose(x, (1, 2, 0))
    gt = gathered[:, :d_model].T
    rel = relative_pos_embed[:seq_len]

    bbl = 256
    out_t = pl.pallas_call(
        _add_body,
        grid=(batch // bbl,),
        in_specs=[
            pl.BlockSpec((d_model, bbl), lambda j: (0, j)),
            pl.BlockSpec((seq_len, d_model), lambda j: (0, 0)),
            pl.BlockSpec((seq_len, d_model, bbl), lambda j: (0, 0, j)),
        ],
        out_specs=pl.BlockSpec((seq_len, d_model, bbl), lambda j: (0, 0, j)),
        out_shape=jax.ShapeDtypeStruct((seq_len, d_model, batch), jnp.float32),
        compiler_params=pltpu.CompilerParams(
            dimension_semantics=("arbitrary",),
            vmem_limit_bytes=100 * 1024 * 1024,
        ),
    )(gt, rel, xt)

    return jnp.transpose(out_t, (2, 0, 1))
